# baseline (speedup 1.0000x reference)
import functools

import jax
import jax.numpy as jnp
from jax import lax
from jax.experimental import pallas as pl
from jax.experimental.pallas import tpu as pltpu
from jax.experimental.pallas import tpu_sc as plsc

_BATCH = 16384
_ED = 32
_NC = 2
_NS = 16
_NW = _NC * _NS
_BPW = _BATCH // _NW


def _gather_body(uid_hbm, mid_hbm, utab_hbm, mtab_hbm, uout_hbm, mout_hbm,
                 uidx_v, midx_v, urows_v, mrows_v, usem, msem):
  wid = lax.axis_index("s") * _NC + lax.axis_index("c")
  base = wid * _BPW
  pltpu.sync_copy(uid_hbm.at[pl.ds(base, _BPW)], uidx_v)
  pltpu.sync_copy(mid_hbm.at[pl.ds(base, _BPW)], midx_v)
  cu = pltpu.async_copy(utab_hbm.at[uidx_v], urows_v, usem)
  cm = pltpu.async_copy(mtab_hbm.at[midx_v], mrows_v, msem)
  cu.wait()
  cm.wait()
  pltpu.sync_copy(urows_v, uout_hbm.at[pl.ds(base, _BPW)])
  pltpu.sync_copy(mrows_v, mout_hbm.at[pl.ds(base, _BPW)])


_gather = pl.kernel(
    _gather_body,
    mesh=plsc.VectorSubcoreMesh(core_axis_name="c", subcore_axis_name="s"),
    out_type=[
        jax.ShapeDtypeStruct((_BATCH, _ED), jnp.float32),
        jax.ShapeDtypeStruct((_BATCH, _ED), jnp.float32),
    ],
    scratch_types=[
        pltpu.VMEM((_BPW,), jnp.int32),
        pltpu.VMEM((_BPW,), jnp.int32),
        pltpu.VMEM((_BPW, _ED), jnp.float32),
        pltpu.VMEM((_BPW, _ED), jnp.float32),
        pltpu.SemaphoreType.DMA,
        pltpu.SemaphoreType.DMA,
    ],
    compiler_params=pltpu.CompilerParams(use_tc_tiling_on_sc=False),
)


_BLK = 2048


def _mlp_body(ue_ref, me_ref, w1u_ref, w1m_ref, b1_ref, w2_ref, b2_ref,
              w3_ref, b3_ref, out_ref):
  h = (jnp.dot(ue_ref[...], w1u_ref[...], preferred_element_type=jnp.float32)
       + jnp.dot(me_ref[...], w1m_ref[...], preferred_element_type=jnp.float32)
       + b1_ref[...])
  h = jnp.maximum(h, 0.0)
  h = jnp.dot(h, w2_ref[...], preferred_element_type=jnp.float32) + b2_ref[...]
  h = jnp.maximum(h, 0.0)
  out_ref[...] = (
      jnp.dot(h, w3_ref[...], preferred_element_type=jnp.float32) + b3_ref[...])


_mlp = pl.pallas_call(
    _mlp_body,
    grid=(_BATCH // _BLK,),
    in_specs=[
        pl.BlockSpec((_BLK, _ED), lambda i: (i, 0)),
        pl.BlockSpec((_BLK, _ED), lambda i: (i, 0)),
        pl.BlockSpec((_ED, 256), lambda i: (0, 0)),
        pl.BlockSpec((_ED, 256), lambda i: (0, 0)),
        pl.BlockSpec((1, 256), lambda i: (0, 0)),
        pl.BlockSpec((256, 64), lambda i: (0, 0)),
        pl.BlockSpec((1, 64), lambda i: (0, 0)),
        pl.BlockSpec((64, 1), lambda i: (0, 0)),
        pl.BlockSpec((1, 1), lambda i: (0, 0)),
    ],
    out_specs=pl.BlockSpec((_BLK, 1), lambda i: (i, 0)),
    out_shape=jax.ShapeDtypeStruct((_BATCH, 1), jnp.float32),
)


@jax.jit
def kernel(user_id, movie_title, user_table, movie_table, W1, b1, W2, b2, W3,
           b3):
  ue, me = _gather(user_id.astype(jnp.int32), movie_title.astype(jnp.int32),
                   user_table, movie_table)
  return _mlp(ue, me, W1[:_ED], W1[_ED:], b1.reshape(1, 256), W2,
              b2.reshape(1, 64), W3, b3.reshape(1, 1))


# speedup vs baseline: 1.0500x; 1.0500x over previous
import jax
import jax.numpy as jnp
from jax import lax
from jax.experimental import pallas as pl
from jax.experimental.pallas import tpu as pltpu
from jax.experimental.pallas import tpu_sc as plsc

_BATCH = 16384
_ED = 32
_NC = 2
_NS = 16
_NW = _NC * _NS
_BPW = _BATCH // _NW
_CHUNK = 128
_NCH = _BPW // _CHUNK

_PBLK = 512
_UG = 489 * _PBLK
_MG = 49 * _PBLK


def _pack_body(x0_ref, x1_ref, x2_ref, x3_ref, out_ref):
  eye = jnp.eye(_ED, dtype=jnp.float32)
  for a, x_ref in enumerate((x0_ref, x1_ref, x2_ref, x3_ref)):
    out_ref[:, _ED * a:_ED * (a + 1)] = jax.lax.dot_general(
        x_ref[...], eye, (((0,), (0,)), ((), ())),
        preferred_element_type=jnp.float32)


def _make_pack(n_grid, n_cols):
  last = (n_cols - 1) // _PBLK

  def spec(a):
    return pl.BlockSpec(
        (_ED, _PBLK), lambda i, a=a: (0, jnp.minimum(a * n_grid + i, last)))
  return pl.pallas_call(
      _pack_body,
      grid=(n_grid,),
      in_specs=[spec(0), spec(1), spec(2), spec(3)],
      out_specs=pl.BlockSpec((_PBLK, 128), lambda i: (i, 0)),
      out_shape=jax.ShapeDtypeStruct((n_grid * _PBLK, 128), jnp.float32),
  )


_pack_u = _make_pack(_UG // _PBLK, 1000001)
_pack_m = _make_pack(_MG // _PBLK, 100001)


def _gather_body(ug_hbm, mg_hbm, utab_hbm, mtab_hbm, uout_hbm, mout_hbm,
                 idx_v, rows_v, sem):
  wid = lax.axis_index("s") * _NC + lax.axis_index("c")
  base = wid * _BPW
  pltpu.sync_copy(ug_hbm.at[wid], idx_v.at[0])
  pltpu.sync_copy(mg_hbm.at[wid], idx_v.at[1])
  for k in range(_NCH):
    pltpu.async_copy(utab_hbm.at[idx_v.at[0, k]],
                     rows_v.at[pl.ds(k * _CHUNK, _CHUNK)], sem)
  for k in range(_NCH):
    pltpu.make_async_copy(utab_hbm.at[idx_v.at[0, k]],
                          rows_v.at[pl.ds(k * _CHUNK, _CHUNK)], sem).wait()
  pltpu.sync_copy(rows_v, uout_hbm.at[pl.ds(base, _BPW)])
  for k in range(_NCH):
    pltpu.async_copy(mtab_hbm.at[idx_v.at[1, k]],
                     rows_v.at[pl.ds(k * _CHUNK, _CHUNK)], sem)
  for k in range(_NCH):
    pltpu.make_async_copy(mtab_hbm.at[idx_v.at[1, k]],
                          rows_v.at[pl.ds(k * _CHUNK, _CHUNK)], sem).wait()
  pltpu.sync_copy(rows_v, mout_hbm.at[pl.ds(base, _BPW)])


_gather = pl.kernel(
    _gather_body,
    mesh=plsc.VectorSubcoreMesh(core_axis_name="c", subcore_axis_name="s"),
    out_type=[
        jax.ShapeDtypeStruct((_BATCH, 128), jnp.float32),
        jax.ShapeDtypeStruct((_BATCH, 128), jnp.float32),
    ],
    scratch_types=[
        pltpu.VMEM((2, _NCH, _CHUNK), jnp.int32),
        pltpu.VMEM((_BPW, 128), jnp.float32),
        pltpu.SemaphoreType.DMA,
    ],
    compiler_params=pltpu.CompilerParams(use_tc_tiling_on_sc=True),
)


_BLK = 2048


def _mlp_body(ue128_ref, me128_ref, usel_ref, msel_ref,
              w1u_ref, w1m_ref, b1_ref, w2_ref, b2_ref, w3_ref, b3_ref,
              out_ref):
  usel = usel_ref[...]
  msel = msel_ref[...]
  ue = jnp.zeros((_BLK, _ED), jnp.float32)
  me = jnp.zeros((_BLK, _ED), jnp.float32)
  for a in range(4):
    ue = ue + jnp.where(usel == a, ue128_ref[:, a * _ED:(a + 1) * _ED], 0.0)
    me = me + jnp.where(msel == a, me128_ref[:, a * _ED:(a + 1) * _ED], 0.0)
  h = (jnp.dot(ue, w1u_ref[...], preferred_element_type=jnp.float32)
       + jnp.dot(me, w1m_ref[...], preferred_element_type=jnp.float32)
       + b1_ref[...])
  h = jnp.maximum(h, 0.0)
  h = jnp.dot(h, w2_ref[...], preferred_element_type=jnp.float32) + b2_ref[...]
  h = jnp.maximum(h, 0.0)
  out_ref[...] = (
      jnp.dot(h, w3_ref[...], preferred_element_type=jnp.float32) + b3_ref[...])


_mlp = pl.pallas_call(
    _mlp_body,
    grid=(_BATCH // _BLK,),
    in_specs=[
        pl.BlockSpec((_BLK, 128), lambda i: (i, 0)),
        pl.BlockSpec((_BLK, 128), lambda i: (i, 0)),
        pl.BlockSpec((_BLK, 1), lambda i: (i, 0)),
        pl.BlockSpec((_BLK, 1), lambda i: (i, 0)),
        pl.BlockSpec((_ED, 256), lambda i: (0, 0)),
        pl.BlockSpec((_ED, 256), lambda i: (0, 0)),
        pl.BlockSpec((1, 256), lambda i: (0, 0)),
        pl.BlockSpec((256, 64), lambda i: (0, 0)),
        pl.BlockSpec((1, 64), lambda i: (0, 0)),
        pl.BlockSpec((64, 1), lambda i: (0, 0)),
        pl.BlockSpec((1, 1), lambda i: (0, 0)),
    ],
    out_specs=pl.BlockSpec((_BLK, 1), lambda i: (i, 0)),
    out_shape=jax.ShapeDtypeStruct((_BATCH, 1), jnp.float32),
)


@jax.jit
def kernel(user_id, movie_title, user_table, movie_table, W1, b1, W2, b2, W3,
           b3):
  uid = user_id.astype(jnp.int32)
  mid = movie_title.astype(jnp.int32)
  ut_p = _pack_u(user_table.T, user_table.T, user_table.T, user_table.T)
  mt_p = _pack_m(movie_table.T, movie_table.T, movie_table.T, movie_table.T)
  ug = (uid % _UG).reshape(_NW, _NCH, _CHUNK)
  mg = (mid % _MG).reshape(_NW, _NCH, _CHUNK)
  ue128, me128 = _gather(ug, mg, ut_p, mt_p)
  usel = (uid // _UG).reshape(_BATCH, 1)
  msel = (mid // _MG).reshape(_BATCH, 1)
  return _mlp(ue128, me128, usel, msel, W1[:_ED], W1[_ED:],
              b1.reshape(1, 256), W2, b2.reshape(1, 64), W3, b3.reshape(1, 1))


# speedup vs baseline: 1.8494x; 1.7613x over previous
import jax
import jax.numpy as jnp
from jax import lax
from jax.experimental import pallas as pl
from jax.experimental.pallas import tpu as pltpu
from jax.experimental.pallas import tpu_sc as plsc

_BATCH = 16384
_ED = 32
_NC = 2
_NS = 16
_NW = _NC * _NS
_BPW = _BATCH // _NW
_CHUNK = 128
_NCH = _BPW // _CHUNK

_PBLK = 1024
_UG = 245 * _PBLK
_MG = 25 * _PBLK


def _pack_body(x0_ref, x1_ref, x2_ref, x3_ref, out_ref):
  xcat = jnp.concatenate(
      [x0_ref[...], x1_ref[...], x2_ref[...], x3_ref[...]], axis=0)
  out_ref[...] = jax.lax.dot_general(
      xcat, jnp.eye(128, dtype=jnp.float32), (((0,), (0,)), ((), ())),
      preferred_element_type=jnp.float32)


def _make_pack(n_grid, n_cols):
  last = (n_cols - 1) // _PBLK

  def spec(a):
    return pl.BlockSpec(
        (_ED, _PBLK), lambda i, a=a: (0, jnp.minimum(a * n_grid + i, last)))
  return pl.pallas_call(
      _pack_body,
      grid=(n_grid,),
      in_specs=[spec(0), spec(1), spec(2), spec(3)],
      out_specs=pl.BlockSpec((_PBLK, 128), lambda i: (i, 0)),
      out_shape=jax.ShapeDtypeStruct((n_grid * _PBLK, 128), jnp.float32),
  )


_pack_u = _make_pack(_UG // _PBLK, 1000001)
_pack_m = _make_pack(_MG // _PBLK, 100001)


def _gather_body(ug_hbm, mg_hbm, utab_hbm, mtab_hbm, uout_hbm, mout_hbm,
                 idx_v, rows_v, sem):
  wid = lax.axis_index("s") * _NC + lax.axis_index("c")
  base = wid * _BPW
  pltpu.sync_copy(ug_hbm.at[wid], idx_v.at[0])
  pltpu.sync_copy(mg_hbm.at[wid], idx_v.at[1])
  for k in range(_NCH):
    pltpu.async_copy(utab_hbm.at[idx_v.at[0, k]],
                     rows_v.at[pl.ds(k * _CHUNK, _CHUNK)], sem)
  for k in range(_NCH):
    pltpu.make_async_copy(utab_hbm.at[idx_v.at[0, k]],
                          rows_v.at[pl.ds(k * _CHUNK, _CHUNK)], sem).wait()
  pltpu.sync_copy(rows_v, uout_hbm.at[pl.ds(base, _BPW)])
  for k in range(_NCH):
    pltpu.async_copy(mtab_hbm.at[idx_v.at[1, k]],
                     rows_v.at[pl.ds(k * _CHUNK, _CHUNK)], sem)
  for k in range(_NCH):
    pltpu.make_async_copy(mtab_hbm.at[idx_v.at[1, k]],
                          rows_v.at[pl.ds(k * _CHUNK, _CHUNK)], sem).wait()
  pltpu.sync_copy(rows_v, mout_hbm.at[pl.ds(base, _BPW)])


_gather = pl.kernel(
    _gather_body,
    mesh=plsc.VectorSubcoreMesh(core_axis_name="c", subcore_axis_name="s"),
    out_type=[
        jax.ShapeDtypeStruct((_BATCH, 128), jnp.float32),
        jax.ShapeDtypeStruct((_BATCH, 128), jnp.float32),
    ],
    scratch_types=[
        pltpu.VMEM((2, _NCH, _CHUNK), jnp.int32),
        pltpu.VMEM((_BPW, 128), jnp.float32),
        pltpu.SemaphoreType.DMA,
    ],
    compiler_params=pltpu.CompilerParams(use_tc_tiling_on_sc=True),
)


_BLK = 2048


def _mlp_body(ue128_ref, me128_ref, usel_ref, msel_ref,
              w1u_ref, w1m_ref, b1_ref, w2_ref, b2_ref, w3_ref, b3_ref,
              out_ref):
  usel = usel_ref[...]
  msel = msel_ref[...]
  ue = jnp.zeros((_BLK, _ED), jnp.float32)
  me = jnp.zeros((_BLK, _ED), jnp.float32)
  for a in range(4):
    ue = ue + jnp.where(usel == a, ue128_ref[:, a * _ED:(a + 1) * _ED], 0.0)
    me = me + jnp.where(msel == a, me128_ref[:, a * _ED:(a + 1) * _ED], 0.0)
  h = (jnp.dot(ue, w1u_ref[...], preferred_element_type=jnp.float32)
       + jnp.dot(me, w1m_ref[...], preferred_element_type=jnp.float32)
       + b1_ref[...])
  h = jnp.maximum(h, 0.0)
  h = jnp.dot(h, w2_ref[...], preferred_element_type=jnp.float32) + b2_ref[...]
  h = jnp.maximum(h, 0.0)
  out_ref[...] = (
      jnp.dot(h, w3_ref[...], preferred_element_type=jnp.float32) + b3_ref[...])


_mlp = pl.pallas_call(
    _mlp_body,
    grid=(_BATCH // _BLK,),
    in_specs=[
        pl.BlockSpec((_BLK, 128), lambda i: (i, 0)),
        pl.BlockSpec((_BLK, 128), lambda i: (i, 0)),
        pl.BlockSpec((_BLK, 1), lambda i: (i, 0)),
        pl.BlockSpec((_BLK, 1), lambda i: (i, 0)),
        pl.BlockSpec((_ED, 256), lambda i: (0, 0)),
        pl.BlockSpec((_ED, 256), lambda i: (0, 0)),
        pl.BlockSpec((1, 256), lambda i: (0, 0)),
        pl.BlockSpec((256, 64), lambda i: (0, 0)),
        pl.BlockSpec((1, 64), lambda i: (0, 0)),
        pl.BlockSpec((64, 1), lambda i: (0, 0)),
        pl.BlockSpec((1, 1), lambda i: (0, 0)),
    ],
    out_specs=pl.BlockSpec((_BLK, 1), lambda i: (i, 0)),
    out_shape=jax.ShapeDtypeStruct((_BATCH, 1), jnp.float32),
)


@jax.jit
def kernel(user_id, movie_title, user_table, movie_table, W1, b1, W2, b2, W3,
           b3):
  uid = user_id.astype(jnp.int32)
  mid = movie_title.astype(jnp.int32)
  ut_p = _pack_u(user_table.T, user_table.T, user_table.T, user_table.T)
  mt_p = _pack_m(movie_table.T, movie_table.T, movie_table.T, movie_table.T)
  ug = (uid % _UG).reshape(_NW, _NCH, _CHUNK)
  mg = (mid % _MG).reshape(_NW, _NCH, _CHUNK)
  ue128, me128 = _gather(ug, mg, ut_p, mt_p)
  usel = (uid // _UG).reshape(_BATCH, 1)
  msel = (mid // _MG).reshape(_BATCH, 1)
  return _mlp(ue128, me128, usel, msel, W1[:_ED], W1[_ED:],
              b1.reshape(1, 256), W2, b2.reshape(1, 64), W3, b3.reshape(1, 1))


# speedup vs baseline: 2.3892x; 1.2919x over previous
import jax
import jax.numpy as jnp
from jax import lax
from jax.experimental import pallas as pl
from jax.experimental.pallas import tpu as pltpu
from jax.experimental.pallas import tpu_sc as plsc

_BATCH = 16384
_ED = 32
_NC = 2
_NS = 16
_NW = _NC * _NS
_BPW = _BATCH // _NW
_CHUNK = 128
_NCH = _BPW // _CHUNK

_PBLK = 2048
_UG = 123 * _PBLK
_MG = 13 * _PBLK


def _pack_body(x0_ref, x1_ref, x2_ref, x3_ref, out_ref):
  xcat = jnp.concatenate(
      [x0_ref[...], x1_ref[...], x2_ref[...], x3_ref[...]], axis=0)
  out_ref[...] = jax.lax.dot_general(
      xcat, jnp.eye(128, dtype=jnp.float32), (((0,), (0,)), ((), ())),
      preferred_element_type=jnp.float32)


def _make_pack(n_grid, n_cols):
  last = (n_cols - 1) // _PBLK

  def spec(a):
    return pl.BlockSpec(
        (_ED, _PBLK), lambda i, a=a: (0, jnp.minimum(a * n_grid + i, last)))
  return pl.pallas_call(
      _pack_body,
      grid=(n_grid,),
      in_specs=[spec(0), spec(1), spec(2), spec(3)],
      out_specs=pl.BlockSpec((_PBLK, 128), lambda i: (i, 0)),
      out_shape=jax.ShapeDtypeStruct((n_grid * _PBLK, 128), jnp.float32),
      compiler_params=pltpu.CompilerParams(
          dimension_semantics=("arbitrary",)),
  )


_pack_u = _make_pack(_UG // _PBLK, 1000001)
_pack_m = _make_pack(_MG // _PBLK, 100001)


def _gather_body(ug_hbm, mg_hbm, utab_hbm, mtab_hbm, uout_hbm, mout_hbm,
                 idx_v, rows_v, sem):
  wid = lax.axis_index("s") * _NC + lax.axis_index("c")
  base = wid * _BPW
  pltpu.sync_copy(ug_hbm.at[wid], idx_v.at[0])
  pltpu.sync_copy(mg_hbm.at[wid], idx_v.at[1])
  for k in range(_NCH):
    pltpu.async_copy(utab_hbm.at[idx_v.at[0, k]],
                     rows_v.at[pl.ds(k * _CHUNK, _CHUNK)], sem)
  for k in range(_NCH):
    pltpu.make_async_copy(utab_hbm.at[idx_v.at[0, k]],
                          rows_v.at[pl.ds(k * _CHUNK, _CHUNK)], sem).wait()
  pltpu.sync_copy(rows_v, uout_hbm.at[pl.ds(base, _BPW)])
  for k in range(_NCH):
    pltpu.async_copy(mtab_hbm.at[idx_v.at[1, k]],
                     rows_v.at[pl.ds(k * _CHUNK, _CHUNK)], sem)
  for k in range(_NCH):
    pltpu.make_async_copy(mtab_hbm.at[idx_v.at[1, k]],
                          rows_v.at[pl.ds(k * _CHUNK, _CHUNK)], sem).wait()
  pltpu.sync_copy(rows_v, mout_hbm.at[pl.ds(base, _BPW)])


_gather = pl.kernel(
    _gather_body,
    mesh=plsc.VectorSubcoreMesh(core_axis_name="c", subcore_axis_name="s"),
    out_type=[
        jax.ShapeDtypeStruct((_BATCH, 128), jnp.float32),
        jax.ShapeDtypeStruct((_BATCH, 128), jnp.float32),
    ],
    scratch_types=[
        pltpu.VMEM((2, _NCH, _CHUNK), jnp.int32),
        pltpu.VMEM((_BPW, 128), jnp.float32),
        pltpu.SemaphoreType.DMA,
    ],
    compiler_params=pltpu.CompilerParams(use_tc_tiling_on_sc=True),
)


_BLK = 2048


def _mlp_body(ue128_ref, me128_ref, usel_ref, msel_ref,
              w1u_ref, w1m_ref, b1_ref, w2_ref, b2_ref, w3_ref, b3_ref,
              out_ref):
  usel = usel_ref[...]
  msel = msel_ref[...]
  ue = jnp.zeros((_BLK, _ED), jnp.float32)
  me = jnp.zeros((_BLK, _ED), jnp.float32)
  for a in range(4):
    ue = ue + jnp.where(usel == a, ue128_ref[:, a * _ED:(a + 1) * _ED], 0.0)
    me = me + jnp.where(msel == a, me128_ref[:, a * _ED:(a + 1) * _ED], 0.0)
  h = (jnp.dot(ue, w1u_ref[...], preferred_element_type=jnp.float32)
       + jnp.dot(me, w1m_ref[...], preferred_element_type=jnp.float32)
       + b1_ref[...])
  h = jnp.maximum(h, 0.0)
  h = jnp.dot(h, w2_ref[...], preferred_element_type=jnp.float32) + b2_ref[...]
  h = jnp.maximum(h, 0.0)
  out_ref[...] = (
      jnp.dot(h, w3_ref[...], preferred_element_type=jnp.float32) + b3_ref[...])


_mlp = pl.pallas_call(
    _mlp_body,
    grid=(_BATCH // _BLK,),
    in_specs=[
        pl.BlockSpec((_BLK, 128), lambda i: (i, 0)),
        pl.BlockSpec((_BLK, 128), lambda i: (i, 0)),
        pl.BlockSpec((_BLK, 1), lambda i: (i, 0)),
        pl.BlockSpec((_BLK, 1), lambda i: (i, 0)),
        pl.BlockSpec((_ED, 256), lambda i: (0, 0)),
        pl.BlockSpec((_ED, 256), lambda i: (0, 0)),
        pl.BlockSpec((1, 256), lambda i: (0, 0)),
        pl.BlockSpec((256, 64), lambda i: (0, 0)),
        pl.BlockSpec((1, 64), lambda i: (0, 0)),
        pl.BlockSpec((64, 1), lambda i: (0, 0)),
        pl.BlockSpec((1, 1), lambda i: (0, 0)),
    ],
    out_specs=pl.BlockSpec((_BLK, 1), lambda i: (i, 0)),
    out_shape=jax.ShapeDtypeStruct((_BATCH, 1), jnp.float32),
)


@jax.jit
def kernel(user_id, movie_title, user_table, movie_table, W1, b1, W2, b2, W3,
           b3):
  uid = user_id.astype(jnp.int32)
  mid = movie_title.astype(jnp.int32)
  ut_p = _pack_u(user_table.T, user_table.T, user_table.T, user_table.T)
  mt_p = _pack_m(movie_table.T, movie_table.T, movie_table.T, movie_table.T)
  ug = (uid % _UG).reshape(_NW, _NCH, _CHUNK)
  mg = (mid % _MG).reshape(_NW, _NCH, _CHUNK)
  ue128, me128 = _gather(ug, mg, ut_p, mt_p)
  usel = (uid // _UG).reshape(_BATCH, 1)
  msel = (mid // _MG).reshape(_BATCH, 1)
  return _mlp(ue128, me128, usel, msel, W1[:_ED], W1[_ED:],
              b1.reshape(1, 256), W2, b2.reshape(1, 64), W3, b3.reshape(1, 1))


# speedup vs baseline: 2.9179x; 1.2213x over previous
import jax
import jax.numpy as jnp
from jax import lax
from jax.experimental import pallas as pl
from jax.experimental.pallas import tpu as pltpu
from jax.experimental.pallas import tpu_sc as plsc

_BATCH = 16384
_ED = 32
_NC = 2
_NS = 16
_NW = _NC * _NS
_BPW = _BATCH // _NW
_CHUNK = 128
_NCH = _BPW // _CHUNK

_PBLK = 4096
_UG = 62 * _PBLK
_MG = 7 * _PBLK


def _pack_body(x0_ref, x1_ref, x2_ref, x3_ref, out_ref):
  xcat = jnp.concatenate(
      [x0_ref[...], x1_ref[...], x2_ref[...], x3_ref[...]], axis=0)
  out_ref[...] = jax.lax.dot_general(
      xcat, jnp.eye(128, dtype=jnp.float32), (((0,), (0,)), ((), ())),
      preferred_element_type=jnp.float32)


def _make_pack(n_grid, n_cols):
  last = (n_cols - 1) // _PBLK

  def spec(a):
    return pl.BlockSpec(
        (_ED, _PBLK), lambda i, a=a: (0, jnp.minimum(a * n_grid + i, last)))
  return pl.pallas_call(
      _pack_body,
      grid=(n_grid,),
      in_specs=[spec(0), spec(1), spec(2), spec(3)],
      out_specs=pl.BlockSpec((_PBLK, 128), lambda i: (i, 0)),
      out_shape=jax.ShapeDtypeStruct((n_grid * _PBLK, 128), jnp.float32),
      compiler_params=pltpu.CompilerParams(
          dimension_semantics=("arbitrary",)),
  )


_pack_u = _make_pack(_UG // _PBLK, 1000001)
_pack_m = _make_pack(_MG // _PBLK, 100001)


def _gather_body(ug_hbm, mg_hbm, utab_hbm, mtab_hbm, uout_hbm, mout_hbm,
                 idx_v, rows_v, sem):
  wid = lax.axis_index("s") * _NC + lax.axis_index("c")
  base = wid * _BPW
  pltpu.sync_copy(ug_hbm.at[wid], idx_v.at[0])
  pltpu.sync_copy(mg_hbm.at[wid], idx_v.at[1])
  for k in range(_NCH):
    pltpu.async_copy(utab_hbm.at[idx_v.at[0, k]],
                     rows_v.at[pl.ds(k * _CHUNK, _CHUNK)], sem)
  for k in range(_NCH):
    pltpu.make_async_copy(utab_hbm.at[idx_v.at[0, k]],
                          rows_v.at[pl.ds(k * _CHUNK, _CHUNK)], sem).wait()
  pltpu.sync_copy(rows_v, uout_hbm.at[pl.ds(base, _BPW)])
  for k in range(_NCH):
    pltpu.async_copy(mtab_hbm.at[idx_v.at[1, k]],
                     rows_v.at[pl.ds(k * _CHUNK, _CHUNK)], sem)
  for k in range(_NCH):
    pltpu.make_async_copy(mtab_hbm.at[idx_v.at[1, k]],
                          rows_v.at[pl.ds(k * _CHUNK, _CHUNK)], sem).wait()
  pltpu.sync_copy(rows_v, mout_hbm.at[pl.ds(base, _BPW)])


_gather = pl.kernel(
    _gather_body,
    mesh=plsc.VectorSubcoreMesh(core_axis_name="c", subcore_axis_name="s"),
    out_type=[
        jax.ShapeDtypeStruct((_BATCH, 128), jnp.float32),
        jax.ShapeDtypeStruct((_BATCH, 128), jnp.float32),
    ],
    scratch_types=[
        pltpu.VMEM((2, _NCH, _CHUNK), jnp.int32),
        pltpu.VMEM((_BPW, 128), jnp.float32),
        pltpu.SemaphoreType.DMA,
    ],
    compiler_params=pltpu.CompilerParams(use_tc_tiling_on_sc=True),
)


_BLK = 2048


def _mlp_body(ue128_ref, me128_ref, usel_ref, msel_ref,
              w1u_ref, w1m_ref, b1_ref, w2_ref, b2_ref, w3_ref, b3_ref,
              out_ref):
  usel = usel_ref[...]
  msel = msel_ref[...]
  ue = jnp.zeros((_BLK, _ED), jnp.float32)
  me = jnp.zeros((_BLK, _ED), jnp.float32)
  for a in range(4):
    ue = ue + jnp.where(usel == a, ue128_ref[:, a * _ED:(a + 1) * _ED], 0.0)
    me = me + jnp.where(msel == a, me128_ref[:, a * _ED:(a + 1) * _ED], 0.0)
  h = (jnp.dot(ue, w1u_ref[...], preferred_element_type=jnp.float32)
       + jnp.dot(me, w1m_ref[...], preferred_element_type=jnp.float32)
       + b1_ref[...])
  h = jnp.maximum(h, 0.0)
  h = jnp.dot(h, w2_ref[...], preferred_element_type=jnp.float32) + b2_ref[...]
  h = jnp.maximum(h, 0.0)
  out_ref[...] = (
      jnp.dot(h, w3_ref[...], preferred_element_type=jnp.float32) + b3_ref[...])


_mlp = pl.pallas_call(
    _mlp_body,
    grid=(_BATCH // _BLK,),
    in_specs=[
        pl.BlockSpec((_BLK, 128), lambda i: (i, 0)),
        pl.BlockSpec((_BLK, 128), lambda i: (i, 0)),
        pl.BlockSpec((_BLK, 1), lambda i: (i, 0)),
        pl.BlockSpec((_BLK, 1), lambda i: (i, 0)),
        pl.BlockSpec((_ED, 256), lambda i: (0, 0)),
        pl.BlockSpec((_ED, 256), lambda i: (0, 0)),
        pl.BlockSpec((1, 256), lambda i: (0, 0)),
        pl.BlockSpec((256, 64), lambda i: (0, 0)),
        pl.BlockSpec((1, 64), lambda i: (0, 0)),
        pl.BlockSpec((64, 1), lambda i: (0, 0)),
        pl.BlockSpec((1, 1), lambda i: (0, 0)),
    ],
    out_specs=pl.BlockSpec((_BLK, 1), lambda i: (i, 0)),
    out_shape=jax.ShapeDtypeStruct((_BATCH, 1), jnp.float32),
)


@jax.jit
def kernel(user_id, movie_title, user_table, movie_table, W1, b1, W2, b2, W3,
           b3):
  uid = user_id.astype(jnp.int32)
  mid = movie_title.astype(jnp.int32)
  ut_p = _pack_u(user_table.T, user_table.T, user_table.T, user_table.T)
  mt_p = _pack_m(movie_table.T, movie_table.T, movie_table.T, movie_table.T)
  ug = (uid % _UG).reshape(_NW, _NCH, _CHUNK)
  mg = (mid % _MG).reshape(_NW, _NCH, _CHUNK)
  ue128, me128 = _gather(ug, mg, ut_p, mt_p)
  usel = (uid // _UG).reshape(_BATCH, 1)
  msel = (mid // _MG).reshape(_BATCH, 1)
  return _mlp(ue128, me128, usel, msel, W1[:_ED], W1[_ED:],
              b1.reshape(1, 256), W2, b2.reshape(1, 64), W3, b3.reshape(1, 1))


# speedup vs baseline: 3.1485x; 1.0790x over previous
import jax
import jax.numpy as jnp
from jax import lax
from jax.experimental import pallas as pl
from jax.experimental.pallas import tpu as pltpu
from jax.experimental.pallas import tpu_sc as plsc

_BATCH = 16384
_ED = 32
_NC = 2
_NS = 16
_NW = _NC * _NS
_BPW = _BATCH // _NW
_CHUNK = 128
_NCH = _BPW // _CHUNK

_PBLK = 8192
_UG = 31 * _PBLK
_MG = 4 * _PBLK


def _pack_body(x0_ref, x1_ref, x2_ref, x3_ref, out_ref):
  xcat = jnp.concatenate(
      [x0_ref[...], x1_ref[...], x2_ref[...], x3_ref[...]], axis=0)
  out_ref[...] = jax.lax.dot_general(
      xcat, jnp.eye(128, dtype=jnp.float32), (((0,), (0,)), ((), ())),
      preferred_element_type=jnp.float32)


def _make_pack(n_grid, n_cols):
  last = (n_cols - 1) // _PBLK

  def spec(a):
    return pl.BlockSpec(
        (_ED, _PBLK), lambda i, a=a: (0, jnp.minimum(a * n_grid + i, last)))
  return pl.pallas_call(
      _pack_body,
      grid=(n_grid,),
      in_specs=[spec(0), spec(1), spec(2), spec(3)],
      out_specs=pl.BlockSpec((_PBLK, 128), lambda i: (i, 0)),
      out_shape=jax.ShapeDtypeStruct((n_grid * _PBLK, 128), jnp.float32),
      compiler_params=pltpu.CompilerParams(
          dimension_semantics=("arbitrary",)),
  )


_pack_u = _make_pack(_UG // _PBLK, 1000001)
_pack_m = _make_pack(_MG // _PBLK, 100001)


def _gather_body(ug_hbm, mg_hbm, utab_hbm, mtab_hbm, uout_hbm, mout_hbm,
                 idx_v, rows_v, sem):
  wid = lax.axis_index("s") * _NC + lax.axis_index("c")
  base = wid * _BPW
  pltpu.sync_copy(ug_hbm.at[wid], idx_v.at[0])
  pltpu.sync_copy(mg_hbm.at[wid], idx_v.at[1])
  for k in range(_NCH):
    pltpu.async_copy(utab_hbm.at[idx_v.at[0, k]],
                     rows_v.at[pl.ds(k * _CHUNK, _CHUNK)], sem)
  for k in range(_NCH):
    pltpu.make_async_copy(utab_hbm.at[idx_v.at[0, k]],
                          rows_v.at[pl.ds(k * _CHUNK, _CHUNK)], sem).wait()
  pltpu.sync_copy(rows_v, uout_hbm.at[pl.ds(base, _BPW)])
  for k in range(_NCH):
    pltpu.async_copy(mtab_hbm.at[idx_v.at[1, k]],
                     rows_v.at[pl.ds(k * _CHUNK, _CHUNK)], sem)
  for k in range(_NCH):
    pltpu.make_async_copy(mtab_hbm.at[idx_v.at[1, k]],
                          rows_v.at[pl.ds(k * _CHUNK, _CHUNK)], sem).wait()
  pltpu.sync_copy(rows_v, mout_hbm.at[pl.ds(base, _BPW)])


_gather = pl.kernel(
    _gather_body,
    mesh=plsc.VectorSubcoreMesh(core_axis_name="c", subcore_axis_name="s"),
    out_type=[
        jax.ShapeDtypeStruct((_BATCH, 128), jnp.float32),
        jax.ShapeDtypeStruct((_BATCH, 128), jnp.float32),
    ],
    scratch_types=[
        pltpu.VMEM((2, _NCH, _CHUNK), jnp.int32),
        pltpu.VMEM((_BPW, 128), jnp.float32),
        pltpu.SemaphoreType.DMA,
    ],
    compiler_params=pltpu.CompilerParams(use_tc_tiling_on_sc=True),
)


_BLK = 2048


def _mlp_body(ue128_ref, me128_ref, usel_ref, msel_ref,
              w1u_ref, w1m_ref, b1_ref, w2_ref, b2_ref, w3_ref, b3_ref,
              out_ref):
  usel = usel_ref[...]
  msel = msel_ref[...]
  ue = jnp.zeros((_BLK, _ED), jnp.float32)
  me = jnp.zeros((_BLK, _ED), jnp.float32)
  for a in range(4):
    ue = ue + jnp.where(usel == a, ue128_ref[:, a * _ED:(a + 1) * _ED], 0.0)
    me = me + jnp.where(msel == a, me128_ref[:, a * _ED:(a + 1) * _ED], 0.0)
  h = (jnp.dot(ue, w1u_ref[...], preferred_element_type=jnp.float32)
       + jnp.dot(me, w1m_ref[...], preferred_element_type=jnp.float32)
       + b1_ref[...])
  h = jnp.maximum(h, 0.0)
  h = jnp.dot(h, w2_ref[...], preferred_element_type=jnp.float32) + b2_ref[...]
  h = jnp.maximum(h, 0.0)
  out_ref[...] = (
      jnp.dot(h, w3_ref[...], preferred_element_type=jnp.float32) + b3_ref[...])


_mlp = pl.pallas_call(
    _mlp_body,
    grid=(_BATCH // _BLK,),
    in_specs=[
        pl.BlockSpec((_BLK, 128), lambda i: (i, 0)),
        pl.BlockSpec((_BLK, 128), lambda i: (i, 0)),
        pl.BlockSpec((_BLK, 1), lambda i: (i, 0)),
        pl.BlockSpec((_BLK, 1), lambda i: (i, 0)),
        pl.BlockSpec((_ED, 256), lambda i: (0, 0)),
        pl.BlockSpec((_ED, 256), lambda i: (0, 0)),
        pl.BlockSpec((1, 256), lambda i: (0, 0)),
        pl.BlockSpec((256, 64), lambda i: (0, 0)),
        pl.BlockSpec((1, 64), lambda i: (0, 0)),
        pl.BlockSpec((64, 1), lambda i: (0, 0)),
        pl.BlockSpec((1, 1), lambda i: (0, 0)),
    ],
    out_specs=pl.BlockSpec((_BLK, 1), lambda i: (i, 0)),
    out_shape=jax.ShapeDtypeStruct((_BATCH, 1), jnp.float32),
)


@jax.jit
def kernel(user_id, movie_title, user_table, movie_table, W1, b1, W2, b2, W3,
           b3):
  uid = user_id.astype(jnp.int32)
  mid = movie_title.astype(jnp.int32)
  ut_p = _pack_u(user_table.T, user_table.T, user_table.T, user_table.T)
  mt_p = _pack_m(movie_table.T, movie_table.T, movie_table.T, movie_table.T)
  ug = (uid % _UG).reshape(_NW, _NCH, _CHUNK)
  mg = (mid % _MG).reshape(_NW, _NCH, _CHUNK)
  ue128, me128 = _gather(ug, mg, ut_p, mt_p)
  usel = (uid // _UG).reshape(_BATCH, 1)
  msel = (mid // _MG).reshape(_BATCH, 1)
  return _mlp(ue128, me128, usel, msel, W1[:_ED], W1[_ED:],
              b1.reshape(1, 256), W2, b2.reshape(1, 64), W3, b3.reshape(1, 1))


# speedup vs baseline: 3.7659x; 1.1961x over previous
import jax
import jax.numpy as jnp
from jax import lax
from jax.experimental import pallas as pl
from jax.experimental.pallas import tpu as pltpu
from jax.experimental.pallas import tpu_sc as plsc

_BATCH = 16384
_ED = 32
_NC = 2
_NS = 16
_NW = _NC * _NS
_BPW = _BATCH // _NW
_CHUNK = 128
_NCH = _BPW // _CHUNK

_PBLK = 8192
_UG = 31 * _PBLK
_MG = 4 * _PBLK


def _pack_body(x0_ref, x1_ref, x2_ref, x3_ref, out_ref):
  xcat = jnp.concatenate(
      [x0_ref[...], x1_ref[...], x2_ref[...], x3_ref[...]], axis=0)
  out_ref[...] = jax.lax.dot_general(
      xcat, jnp.eye(128, dtype=jnp.float32), (((0,), (0,)), ((), ())),
      preferred_element_type=jnp.float32)


def _make_pack(n_grid, n_cols):
  last = (n_cols - 1) // _PBLK

  def spec(a):
    return pl.BlockSpec(
        (_ED, _PBLK), lambda i, a=a: (0, jnp.minimum(a * n_grid + i, last)))
  return pl.pallas_call(
      _pack_body,
      grid=(n_grid,),
      in_specs=[spec(0), spec(1), spec(2), spec(3)],
      out_specs=pl.BlockSpec((_PBLK, 128), lambda i: (i, 0)),
      out_shape=jax.ShapeDtypeStruct((n_grid * _PBLK, 128), jnp.float32),
      compiler_params=pltpu.CompilerParams(
          dimension_semantics=("arbitrary",)),
  )


_pack_u = _make_pack(_UG // _PBLK, 1000001)
_pack_m = _make_pack(_MG // _PBLK, 100001)


def _gather_body(g_hbm, tab_hbm, out_hbm, idx_v, rows_v, sem):
  wid = lax.axis_index("s") * _NC + lax.axis_index("c")
  base = wid * _BPW
  pltpu.sync_copy(g_hbm.at[wid], idx_v)
  for k in range(_NCH):
    pltpu.async_copy(tab_hbm.at[idx_v.at[k]],
                     rows_v.at[pl.ds(k * _CHUNK, _CHUNK)], sem)
  for k in range(_NCH):
    pltpu.make_async_copy(tab_hbm.at[idx_v.at[k]],
                          rows_v.at[pl.ds(k * _CHUNK, _CHUNK)], sem).wait()
  pltpu.sync_copy(rows_v, out_hbm.at[pl.ds(base, _BPW)])


_gather = pl.kernel(
    _gather_body,
    mesh=plsc.VectorSubcoreMesh(core_axis_name="c", subcore_axis_name="s"),
    out_type=jax.ShapeDtypeStruct((_BATCH, 128), jnp.float32),
    scratch_types=[
        pltpu.VMEM((_NCH, _CHUNK), jnp.int32),
        pltpu.VMEM((_BPW, 128), jnp.float32),
        pltpu.SemaphoreType.DMA,
    ],
    compiler_params=pltpu.CompilerParams(use_tc_tiling_on_sc=True),
)


_BLK = 2048


def _mlp_body(ue128_ref, me128_ref, usel_ref, msel_ref,
              w1u4_ref, w1m4_ref, b1_ref, w2_ref, b2_ref, w3_ref, b3_ref,
              out_ref):
  lane_q = jax.lax.broadcasted_iota(jnp.int32, (_BLK, 128), 1) // _ED
  um = jnp.where(lane_q == usel_ref[...], ue128_ref[...], 0.0)
  mm = jnp.where(lane_q == msel_ref[...], me128_ref[...], 0.0)
  h = (jnp.dot(um, w1u4_ref[...], preferred_element_type=jnp.float32)
       + jnp.dot(mm, w1m4_ref[...], preferred_element_type=jnp.float32)
       + b1_ref[...])
  h = jnp.maximum(h, 0.0)
  h = jnp.dot(h, w2_ref[...], preferred_element_type=jnp.float32) + b2_ref[...]
  h = jnp.maximum(h, 0.0)
  out_ref[...] = (
      jax.lax.dot_general(w3_ref[...], h, (((0,), (1,)), ((), ())),
                          preferred_element_type=jnp.float32) + b3_ref[...])


_mlp = pl.pallas_call(
    _mlp_body,
    grid=(_BATCH // _BLK,),
    in_specs=[
        pl.BlockSpec((_BLK, 128), lambda i: (i, 0)),
        pl.BlockSpec((_BLK, 128), lambda i: (i, 0)),
        pl.BlockSpec((_BLK, 1), lambda i: (i, 0)),
        pl.BlockSpec((_BLK, 1), lambda i: (i, 0)),
        pl.BlockSpec((128, 256), lambda i: (0, 0)),
        pl.BlockSpec((128, 256), lambda i: (0, 0)),
        pl.BlockSpec((1, 256), lambda i: (0, 0)),
        pl.BlockSpec((256, 64), lambda i: (0, 0)),
        pl.BlockSpec((1, 64), lambda i: (0, 0)),
        pl.BlockSpec((64, 1), lambda i: (0, 0)),
        pl.BlockSpec((1, 1), lambda i: (0, 0)),
    ],
    out_specs=pl.BlockSpec((1, _BLK), lambda i: (0, i)),
    out_shape=jax.ShapeDtypeStruct((1, _BATCH), jnp.float32),
)


@jax.jit
def kernel(user_id, movie_title, user_table, movie_table, W1, b1, W2, b2, W3,
           b3):
  uid = user_id.astype(jnp.int32)
  mid = movie_title.astype(jnp.int32)
  ut_p = _pack_u(user_table.T, user_table.T, user_table.T, user_table.T)
  mt_p = _pack_m(movie_table.T, movie_table.T, movie_table.T, movie_table.T)
  ug = (uid % _UG).reshape(_NW, _NCH, _CHUNK)
  mg = (mid % _MG).reshape(_NW, _NCH, _CHUNK)
  ue128 = _gather(ug, ut_p)
  me128 = _gather(mg, mt_p)
  usel = (uid // _UG).reshape(_BATCH, 1)
  msel = (mid // _MG).reshape(_BATCH, 1)
  w1u4 = jnp.tile(W1[:_ED], (4, 1))
  w1m4 = jnp.tile(W1[_ED:], (4, 1))
  out = _mlp(ue128, me128, usel, msel, w1u4, w1m4,
             b1.reshape(1, 256), W2, b2.reshape(1, 64), W3, b3.reshape(1, 1))
  return out.reshape(_BATCH, 1)


# speedup vs baseline: 3.7720x; 1.0016x over previous
import jax
import jax.numpy as jnp
from jax import lax
from jax.experimental import pallas as pl
from jax.experimental.pallas import tpu as pltpu
from jax.experimental.pallas import tpu_sc as plsc

_BATCH = 16384
_ED = 32
_NC = 2
_NS = 16
_NW = _NC * _NS
_BPW = _BATCH // _NW
_CHUNK = 128
_NCH = _BPW // _CHUNK

_PBLK = 8192
_UG = 31 * _PBLK
_MG = 4 * _PBLK


def _pack_body(x0_ref, x1_ref, x2_ref, x3_ref, out_ref):
  xcat = jnp.concatenate(
      [x0_ref[...], x1_ref[...], x2_ref[...], x3_ref[...]], axis=0)
  out_ref[...] = jax.lax.dot_general(
      xcat, jnp.eye(128, dtype=jnp.float32), (((0,), (0,)), ((), ())),
      preferred_element_type=jnp.float32)


def _make_pack(n_grid, n_cols):
  last = (n_cols - 1) // _PBLK

  def spec(a):
    return pl.BlockSpec(
        (_ED, _PBLK), lambda i, a=a: (0, jnp.minimum(a * n_grid + i, last)))
  return pl.pallas_call(
      _pack_body,
      grid=(n_grid,),
      in_specs=[spec(0), spec(1), spec(2), spec(3)],
      out_specs=pl.BlockSpec((_PBLK, 128), lambda i: (i, 0)),
      out_shape=jax.ShapeDtypeStruct((n_grid * _PBLK, 128), jnp.float32),
      compiler_params=pltpu.CompilerParams(
          dimension_semantics=("arbitrary",)),
  )


_pack_u = _make_pack(_UG // _PBLK, 1000001)
_pack_m = _make_pack(_MG // _PBLK, 100001)


def _gather_body(g_hbm, tab_hbm, out_hbm, idx_v, rows_v, sem):
  wid = lax.axis_index("s") * _NC + lax.axis_index("c")
  base = wid * _BPW
  pltpu.sync_copy(g_hbm.at[wid], idx_v)
  for k in range(_NCH):
    pltpu.async_copy(tab_hbm.at[idx_v.at[k]],
                     rows_v.at[pl.ds(k * _CHUNK, _CHUNK)], sem)
  for k in range(_NCH):
    pltpu.make_async_copy(tab_hbm.at[idx_v.at[k]],
                          rows_v.at[pl.ds(k * _CHUNK, _CHUNK)], sem).wait()
  pltpu.sync_copy(rows_v, out_hbm.at[pl.ds(base, _BPW)])


_gather = pl.kernel(
    _gather_body,
    mesh=plsc.VectorSubcoreMesh(core_axis_name="c", subcore_axis_name="s"),
    out_type=jax.ShapeDtypeStruct((_BATCH, 128), jnp.float32),
    scratch_types=[
        pltpu.VMEM((_NCH, _CHUNK), jnp.int32),
        pltpu.VMEM((_BPW, 128), jnp.float32),
        pltpu.SemaphoreType.DMA,
    ],
    compiler_params=pltpu.CompilerParams(use_tc_tiling_on_sc=True),
)


_BLK = 2048


def _mlp_body(ue128_ref, me128_ref, usel_ref, msel_ref,
              w1u4_ref, w1m4_ref, b1_ref, w2_ref, b2_ref, w3_ref, b3_ref,
              out_ref):
  lane_q = jax.lax.broadcasted_iota(jnp.int32, (_BLK, 128), 1) // _ED
  um = jnp.where(lane_q == usel_ref[...], ue128_ref[...], 0.0)
  mm = jnp.where(lane_q == msel_ref[...], me128_ref[...], 0.0)
  h = (jnp.dot(um, w1u4_ref[...], preferred_element_type=jnp.float32)
       + jnp.dot(mm, w1m4_ref[...], preferred_element_type=jnp.float32)
       + b1_ref[...])
  h = jnp.maximum(h, 0.0)
  h = jnp.dot(h, w2_ref[...], preferred_element_type=jnp.float32) + b2_ref[...]
  h = jnp.maximum(h, 0.0)
  out_ref[...] = (
      jax.lax.dot_general(w3_ref[...], h, (((0,), (1,)), ((), ())),
                          preferred_element_type=jnp.float32) + b3_ref[...])


_mlp = pl.pallas_call(
    _mlp_body,
    grid=(_BATCH // _BLK,),
    in_specs=[
        pl.BlockSpec((_BLK, 128), lambda i: (i, 0)),
        pl.BlockSpec((_BLK, 128), lambda i: (i, 0)),
        pl.BlockSpec((_BLK, 1), lambda i: (i, 0)),
        pl.BlockSpec((_BLK, 1), lambda i: (i, 0)),
        pl.BlockSpec((128, 256), lambda i: (0, 0)),
        pl.BlockSpec((128, 256), lambda i: (0, 0)),
        pl.BlockSpec((1, 256), lambda i: (0, 0)),
        pl.BlockSpec((256, 64), lambda i: (0, 0)),
        pl.BlockSpec((1, 64), lambda i: (0, 0)),
        pl.BlockSpec((64, 1), lambda i: (0, 0)),
        pl.BlockSpec((1, 1), lambda i: (0, 0)),
    ],
    out_specs=pl.BlockSpec((1, _BLK), lambda i: (0, i)),
    out_shape=jax.ShapeDtypeStruct((1, _BATCH), jnp.float32),
)


@jax.jit
def kernel(user_id, movie_title, user_table, movie_table, W1, b1, W2, b2, W3,
           b3):
  uid = user_id.astype(jnp.int32)
  mid = movie_title.astype(jnp.int32)
  mt_p = _pack_m(movie_table.T, movie_table.T, movie_table.T, movie_table.T)
  mg = (mid % _MG).reshape(_NW, _NCH, _CHUNK)
  me128 = _gather(mg, mt_p)
  ut_p = _pack_u(user_table.T, user_table.T, user_table.T, user_table.T)
  ug = (uid % _UG).reshape(_NW, _NCH, _CHUNK)
  ue128 = _gather(ug, ut_p)
  usel = (uid // _UG).reshape(_BATCH, 1)
  msel = (mid // _MG).reshape(_BATCH, 1)
  w1u4 = jnp.tile(W1[:_ED], (4, 1))
  w1m4 = jnp.tile(W1[_ED:], (4, 1))
  out = _mlp(ue128, me128, usel, msel, w1u4, w1m4,
             b1.reshape(1, 256), W2, b2.reshape(1, 64), W3, b3.reshape(1, 1))
  return out.reshape(_BATCH, 1)


# speedup vs baseline: 4.1718x; 1.1060x over previous
import jax
import jax.numpy as jnp
from jax import lax
from jax.experimental import pallas as pl
from jax.experimental.pallas import tpu as pltpu
from jax.experimental.pallas import tpu_sc as plsc

_BATCH = 16384
_ED = 32
_NC = 2
_NS = 16
_NW = _NC * _NS
_BPW = _BATCH // _NW
_CHUNK = 128
_NCH = _BPW // _CHUNK

_PBLK = 4096
_UG = 31 * _PBLK
_MG = 4 * _PBLK


def _pack_body(*refs):
  out_ref = refs[-1]
  xs = refs[:-1]
  eye = jnp.eye(128, dtype=jnp.float32)
  hi = jnp.concatenate([x[0:16, :] for x in xs], axis=0)
  lo = jnp.concatenate([x[16:32, :] for x in xs], axis=0)
  yh = jax.lax.dot_general(hi, eye, (((0,), (0,)), ((), ())),
                           preferred_element_type=jnp.float32)
  yl = jax.lax.dot_general(lo, eye, (((0,), (0,)), ((), ())),
                           preferred_element_type=jnp.float32)
  uh = jax.lax.bitcast_convert_type(yh.astype(jnp.bfloat16),
                                    jnp.uint16).astype(jnp.uint32)
  ul = jax.lax.bitcast_convert_type(yl.astype(jnp.bfloat16),
                                    jnp.uint16).astype(jnp.uint32)
  out_ref[...] = jax.lax.bitcast_convert_type((uh << 16) | ul, jnp.float32)


def _make_pack(n_grid, n_cols):
  last = (n_cols - 1) // _PBLK

  def spec(a):
    return pl.BlockSpec(
        (_ED, _PBLK), lambda i, a=a: (0, jnp.minimum(a * n_grid + i, last)))
  return pl.pallas_call(
      _pack_body,
      grid=(n_grid,),
      in_specs=[spec(a) for a in range(8)],
      out_specs=pl.BlockSpec((_PBLK, 128), lambda i: (i, 0)),
      out_shape=jax.ShapeDtypeStruct((n_grid * _PBLK, 128), jnp.float32),
      compiler_params=pltpu.CompilerParams(
          dimension_semantics=("arbitrary",)),
  )


_pack_u = _make_pack(_UG // _PBLK, 1000001)
_pack_m = _make_pack(_MG // _PBLK, 100001)


def _gather_body(g_hbm, tab_hbm, out_hbm, idx_v, rows_v, sem):
  wid = lax.axis_index("s") * _NC + lax.axis_index("c")
  base = wid * _BPW
  pltpu.sync_copy(g_hbm.at[wid], idx_v)
  for k in range(_NCH):
    pltpu.async_copy(tab_hbm.at[idx_v.at[k]],
                     rows_v.at[pl.ds(k * _CHUNK, _CHUNK)], sem)
  for k in range(_NCH):
    pltpu.make_async_copy(tab_hbm.at[idx_v.at[k]],
                          rows_v.at[pl.ds(k * _CHUNK, _CHUNK)], sem).wait()
  pltpu.sync_copy(rows_v, out_hbm.at[pl.ds(base, _BPW)])


_gather = pl.kernel(
    _gather_body,
    mesh=plsc.VectorSubcoreMesh(core_axis_name="c", subcore_axis_name="s"),
    out_type=jax.ShapeDtypeStruct((_BATCH, 128), jnp.float32),
    scratch_types=[
        pltpu.VMEM((_NCH, _CHUNK), jnp.int32),
        pltpu.VMEM((_BPW, 128), jnp.float32),
        pltpu.SemaphoreType.DMA,
    ],
    compiler_params=pltpu.CompilerParams(use_tc_tiling_on_sc=True),
)


_BLK = 2048


def _unpack(x, sel):
  lane_q = jax.lax.broadcasted_iota(jnp.int32, (_BLK, 128), 1) // 16
  m = jnp.where(lane_q == sel, x, 0.0)
  ui = jax.lax.bitcast_convert_type(m, jnp.uint32)
  a = jax.lax.bitcast_convert_type(ui & jnp.uint32(0xFFFF0000), jnp.float32)
  b = jax.lax.bitcast_convert_type(ui << 16, jnp.float32)
  return a, b


def _mlp_body(ue128_ref, me128_ref, usel_ref, msel_ref,
              wua_ref, wub_ref, wma_ref, wmb_ref, b1_ref, w2_ref, b2_ref,
              w3_ref, b3_ref, out_ref):
  ua, ub = _unpack(ue128_ref[...], usel_ref[...])
  ma, mb = _unpack(me128_ref[...], msel_ref[...])
  h = (jnp.dot(ua, wua_ref[...], preferred_element_type=jnp.float32)
       + jnp.dot(ub, wub_ref[...], preferred_element_type=jnp.float32)
       + jnp.dot(ma, wma_ref[...], preferred_element_type=jnp.float32)
       + jnp.dot(mb, wmb_ref[...], preferred_element_type=jnp.float32)
       + b1_ref[...])
  h = jnp.maximum(h, 0.0)
  h = jnp.dot(h, w2_ref[...], preferred_element_type=jnp.float32) + b2_ref[...]
  h = jnp.maximum(h, 0.0)
  out_ref[...] = (
      jax.lax.dot_general(w3_ref[...], h, (((0,), (1,)), ((), ())),
                          preferred_element_type=jnp.float32) + b3_ref[...])


_mlp = pl.pallas_call(
    _mlp_body,
    grid=(_BATCH // _BLK,),
    in_specs=[
        pl.BlockSpec((_BLK, 128), lambda i: (i, 0)),
        pl.BlockSpec((_BLK, 128), lambda i: (i, 0)),
        pl.BlockSpec((_BLK, 1), lambda i: (i, 0)),
        pl.BlockSpec((_BLK, 1), lambda i: (i, 0)),
        pl.BlockSpec((128, 256), lambda i: (0, 0)),
        pl.BlockSpec((128, 256), lambda i: (0, 0)),
        pl.BlockSpec((128, 256), lambda i: (0, 0)),
        pl.BlockSpec((128, 256), lambda i: (0, 0)),
        pl.BlockSpec((1, 256), lambda i: (0, 0)),
        pl.BlockSpec((256, 64), lambda i: (0, 0)),
        pl.BlockSpec((1, 64), lambda i: (0, 0)),
        pl.BlockSpec((64, 1), lambda i: (0, 0)),
        pl.BlockSpec((1, 1), lambda i: (0, 0)),
    ],
    out_specs=pl.BlockSpec((1, _BLK), lambda i: (0, i)),
    out_shape=jax.ShapeDtypeStruct((1, _BATCH), jnp.float32),
)


@jax.jit
def kernel(user_id, movie_title, user_table, movie_table, W1, b1, W2, b2, W3,
           b3):
  uid = user_id.astype(jnp.int32)
  mid = movie_title.astype(jnp.int32)
  mt_p = _pack_m(*([movie_table.T] * 8))
  mg = (mid % _MG).reshape(_NW, _NCH, _CHUNK)
  me128 = _gather(mg, mt_p)
  ut_p = _pack_u(*([user_table.T] * 8))
  ug = (uid % _UG).reshape(_NW, _NCH, _CHUNK)
  ue128 = _gather(ug, ut_p)
  usel = (uid // _UG).reshape(_BATCH, 1)
  msel = (mid // _MG).reshape(_BATCH, 1)
  wua = jnp.tile(W1[0:16], (8, 1))
  wub = jnp.tile(W1[16:32], (8, 1))
  wma = jnp.tile(W1[32:48], (8, 1))
  wmb = jnp.tile(W1[48:64], (8, 1))
  out = _mlp(ue128, me128, usel, msel, wua, wub, wma, wmb,
             b1.reshape(1, 256), W2, b2.reshape(1, 64), W3, b3.reshape(1, 1))
  return out.reshape(_BATCH, 1)
